# baseline (speedup 1.0000x reference)
import functools

import jax
import jax.numpy as jnp
from jax import lax
from jax.experimental import pallas as pl
from jax.experimental.pallas import tpu as pltpu
from jax.experimental.pallas import tpu_sc as plsc

N = 10000
E = 160000
H = 128
NT = 16
EPT = E // NT
CH = 80
HCH = CH // 2
NCHUNK = EPT // CH
ROWS_PT = 640


def _prep_body(hin, embw, embb, w1, b1, embew, embeb, c1w, c1b,
               hemb_o, ah_o, db_o, et_o, ce1_o):
    hemb = jnp.dot(hin[...], embw[...], preferred_element_type=jnp.float32) + embb[...]
    hemb_o[...] = hemb
    p = jnp.dot(hemb, w1[...], preferred_element_type=jnp.float32) + b1[...]
    ah = p[:, :H]
    bh = p[:, H:2 * H]
    dh = p[:, 2 * H:3 * H]
    eh = p[:, 3 * H:]
    ah_o[...] = ah
    db_o[0] = jnp.concatenate([dh[:, :64], bh[:, :64]], axis=1)
    db_o[1] = jnp.concatenate([dh[:, 64:], bh[:, 64:]], axis=1)
    et_o[...] = eh
    e0 = embew[...] + embeb[...]
    ce = jnp.dot(e0, c1w[...], preferred_element_type=jnp.float32) + c1b[...]
    ce1_o[...] = jnp.concatenate([ce[:, :64], ce[:, 64:]], axis=0)


def _mid_body(hemb, ah1, nd1, stats, bnh_g, bnh_b, bne_g, bne_b,
              w2, b2, c2w, c2b, embew, embeb,
              h1_o, ah2_o, db2_o, et2_o, ss_o, cb2_o):
    num = jnp.concatenate([nd1[0][:, :64], nd1[1][:, :64]], axis=1)
    den = jnp.concatenate([nd1[0][:, 64:], nd1[1][:, 64:]], axis=1)
    hn = ah1[...] + num / (den + 1e-6)
    m = jnp.mean(hn, axis=0, keepdims=True)
    v = jnp.mean((hn - m) * (hn - m), axis=0, keepdims=True)
    hn = (hn - m) * lax.rsqrt(v + 1e-5) * bnh_g[...] + bnh_b[...]
    h1 = hemb[...] + jnp.maximum(hn, 0.0)
    h1_o[...] = h1
    s1 = jnp.concatenate([stats[0][0:1, :], stats[1][0:1, :]], axis=1)
    q1 = jnp.concatenate([stats[0][1:2, :], stats[1][1:2, :]], axis=1)
    em = s1 / float(E)
    ev = q1 / float(E) - em * em
    scale = bne_g[...] * lax.rsqrt(ev + 1e-5)
    shift = bne_b[...] - em * scale
    ss_o[...] = jnp.concatenate([scale, shift], axis=0)
    e0 = embew[...] + embeb[...]
    cb2_o[...] = (jnp.dot(e0, c2w[...], preferred_element_type=jnp.float32)
                  + c2b[...])
    p2 = jnp.dot(h1, w2[...], preferred_element_type=jnp.float32) + b2[...]
    ah2_o[...] = p2[:, :H]
    bh = p2[:, H:2 * H]
    dh = p2[:, 2 * H:3 * H]
    eh = p2[:, 3 * H:]
    db2_o[0] = jnp.concatenate([dh[:, :64], bh[:, :64]], axis=1)
    db2_o[1] = jnp.concatenate([dh[:, 64:], bh[:, 64:]], axis=1)
    et2_o[...] = eh


def _estats_body(enew, st_o):
    i = pl.program_id(0)
    x = enew[...]
    s = jnp.sum(x, axis=1, keepdims=True)
    q = jnp.sum(x * x, axis=1, keepdims=True)
    cur = jnp.concatenate([s, q], axis=1)

    @pl.when(i == 0)
    def _():
        st_o[...] = cur

    @pl.when(i > 0)
    def _():
        st_o[...] = st_o[...] + cur


def _ce2_body(enew, ss, c2w, cb2, ce_o):
    x = jnp.concatenate([enew[0], enew[1]], axis=1)
    e1 = jnp.maximum(x * ss[0:1, :] + ss[1:2, :], 0.0)
    ce = jnp.dot(e1, c2w[...], preferred_element_type=jnp.float32) + cb2[...]
    ce_o[0] = ce[:, :64]
    ce_o[1] = ce[:, 64:]


def _final_body(h1, ah2, nd2, bnh_g, bnh_b, bih, bhh,
                nw0, nb0, nw1, nb1, nw2, nb2,
                gw0, gb0, gw1, gb1, gw2, gb2,
                node_o, graph_o):
    num = jnp.concatenate([nd2[0][:, :64], nd2[1][:, :64]], axis=1)
    den = jnp.concatenate([nd2[0][:, 64:], nd2[1][:, 64:]], axis=1)
    hn = ah2[...] + num / (den + 1e-6)
    m = jnp.mean(hn, axis=0, keepdims=True)
    v = jnp.mean((hn - m) * (hn - m), axis=0, keepdims=True)
    hn = (hn - m) * lax.rsqrt(v + 1e-5) * bnh_g[...] + bnh_b[...]
    h2 = h1[...] + jnp.maximum(hn, 0.0)
    gates = bih[...] + bhh[...]
    ig = gates[:, :H]
    gg = gates[:, 2 * H:3 * H]
    og = gates[:, 3 * H:]
    c = jax.nn.sigmoid(ig) * jnp.tanh(gg)
    q = jax.nn.sigmoid(og) * jnp.tanh(c)
    scores = jnp.sum(h2 * q, axis=1, keepdims=True)
    mx = jnp.max(scores, axis=0, keepdims=True)
    a = jnp.exp(scores - mx)
    alpha = a / jnp.sum(a, axis=0, keepdims=True)
    r = jnp.sum(alpha * h2, axis=0, keepdims=True)
    q_star = jnp.concatenate([q, r], axis=1)
    x = jnp.maximum(jnp.dot(h2, nw0[...], preferred_element_type=jnp.float32) + nb0[...], 0.0)
    x = jnp.maximum(jnp.dot(x, nw1[...], preferred_element_type=jnp.float32) + nb1[...], 0.0)
    node_o[...] = jnp.dot(x, nw2[...], preferred_element_type=jnp.float32) + nb2[...]
    g = jnp.maximum(jnp.dot(q_star, gw0[...], preferred_element_type=jnp.float32) + gb0[...], 0.0)
    g = jnp.maximum(jnp.dot(g, gw1[...], preferred_element_type=jnp.float32) + gb1[...], 0.0)
    graph_o[...] = jnp.dot(g, gw2[...], preferred_element_type=jnp.float32) + gb2[...]


_MESH = plsc.VectorSubcoreMesh(core_axis_name="c", subcore_axis_name="s",
                               num_cores=2, num_subcores=16)


def _sigmoid16(x):
    return 1.0 / (1.0 + jnp.exp(-x))


def _edge_kernel_body(first_layer, db, et, ce, srcr, dstr,
                      enew_hbm, nd_hbm,
                      sh_nd, src_b, dst_b, db_bufs, e_bufs, aux,
                      buf_ns, ce1_v, semd0, semd1, seme0, seme1,
                      sem3):
    ce_buf = aux
    enew_v = aux
    semd = (semd0, semd1)
    seme = (seme0, seme1)
    cid = lax.axis_index("c")
    sid = lax.axis_index("s")
    coff = cid * N
    ch0 = pl.multiple_of(cid * 64, 64)
    out_base = pl.multiple_of(jnp.minimum(sid * ROWS_PT, N - ROWS_PT), CH)

    @pl.loop(0, CH)
    def _zero(i):
        for g in range(H // 16):
            buf_ns[i, pl.ds(g * 16, 16)] = jnp.zeros((16,), jnp.float32)

    for k in range(ROWS_PT // CH):
        pltpu.sync_copy(buf_ns, sh_nd.at[pl.ds(out_base + k * CH, CH)])

    ce_vecs = None
    if first_layer:
        pltpu.sync_copy(ce.at[cid], ce1_v)
        ce_vecs = [ce1_v[0, pl.ds(g * 16, 16)] for g in range(4)]

    plsc.subcore_barrier()

    def _load_idx(c, slot):
        row = jnp.minimum(sid * NCHUNK + c, NT * NCHUNK - 1)
        pltpu.sync_copy(srcr.at[pl.ds(row, 1)], src_b.at[pl.ds(slot, 1)])
        pltpu.sync_copy(dstr.at[pl.ds(row, 1)], dst_b.at[pl.ds(slot, 1)])
        for g in range(CH // 16):
            sl = pl.ds(g * 16, 16)
            src_b[slot, sl] = src_b[slot, sl] + coff

    def _issue_gather(slot, h):
        idx_s = src_b.at[slot, pl.ds(h * HCH, HCH)]
        idx_d = dst_b.at[slot, pl.ds(h * HCH, HCH)]
        pltpu.async_copy(db.at[idx_s], db_bufs.at[h], semd[h])
        pltpu.async_copy(et.at[idx_d], e_bufs.at[h], seme[h])

    def _wait_gather(h):
        pltpu.make_async_copy(db.at[pl.ds(0, HCH)], db_bufs.at[h], semd[h]).wait()
        pltpu.make_async_copy(et.at[pl.ds(0, HCH)], e_bufs.at[h], seme[h]).wait()

    def _half_compute(h):
        hoff = h * HCH

        def _row(r):
            for g in range(4):
                sl = pl.ds(g * 16, 16)
                sh = pl.ds(64 + g * 16, 16)
                d = db_bufs[h, r, sl]
                b = db_bufs[h, r, sh]
                ev = e_bufs[h, r, pl.ds(ch0 + g * 16, 16)]
                if first_layer:
                    x = ce_vecs[g] + d + ev
                else:
                    x = ce_buf[hoff + r, sl] + d + ev
                s = _sigmoid16(x)
                buf_ns[hoff + r, sl] = s * b
                buf_ns[hoff + r, sh] = s
                if first_layer:
                    enew_v[hoff + r, sl] = x

        plsc.parallel_loop(0, HCH, unroll=8)(_row)

    def _do_chunk(c):
        cs = lax.rem(c, 2)
        ns = 1 - cs
        ebase = sid * EPT + c * CH
        _issue_gather(cs, 1)
        if not first_layer:
            pltpu.async_copy(ce.at[pl.ds(cid * E + ebase, CH)], ce_buf, sem3)
        _load_idx(c + 1, ns)
        if first_layer:
            @pl.when(c > 0)
            def _():
                pltpu.make_async_copy(
                    enew_v, enew_hbm.at[pl.ds(0, CH)], sem3).wait()
        else:
            pltpu.make_async_copy(
                ce.at[pl.ds(0, CH)], ce_buf, sem3).wait()
        _wait_gather(0)
        _half_compute(0)
        _issue_gather(ns, 0)
        _wait_gather(1)
        _half_compute(1)
        pltpu.sync_copy(buf_ns, sh_nd.at[dst_b.at[cs]], add=True)
        if first_layer:
            pltpu.async_copy(enew_v, enew_hbm.at[pl.ds(cid * E + ebase, CH)],
                             sem3)

    _load_idx(0, 0)
    _issue_gather(0, 0)

    @pl.loop(0, NCHUNK)
    def _chunks(c):
        _do_chunk(c)

    if first_layer:
        pltpu.make_async_copy(enew_v, enew_hbm.at[pl.ds(0, CH)], sem3).wait()

    _wait_gather(0)

    plsc.subcore_barrier()
    pltpu.sync_copy(sh_nd.at[pl.ds(out_base, ROWS_PT)],
                    nd_hbm.at[pl.ds(coff + out_base, ROWS_PT)])


def _make_edge_kernel(first_layer):
    outs = [jax.ShapeDtypeStruct((2 * E, 64), jnp.float32),
            jax.ShapeDtypeStruct((2 * N, H), jnp.float32)]
    scratch = [
        pltpu.VMEM_SHARED((N, H), jnp.float32),
        pltpu.VMEM((2, CH), jnp.int32),
        pltpu.VMEM((2, CH), jnp.int32),
        pltpu.VMEM((2, HCH, H), jnp.float32),
        pltpu.VMEM((2, HCH, H), jnp.float32),
        pltpu.VMEM((CH, 64), jnp.float32),
        pltpu.VMEM((CH, H), jnp.float32),
        pltpu.VMEM((1, 64), jnp.float32),
        pltpu.SemaphoreType.DMA,
        pltpu.SemaphoreType.DMA,
        pltpu.SemaphoreType.DMA,
        pltpu.SemaphoreType.DMA,
        pltpu.SemaphoreType.DMA,
    ]
    return pl.kernel(functools.partial(_edge_kernel_body, first_layer),
                     out_type=outs, mesh=_MESH, scratch_types=scratch)


def kernel(h, e, edge_index, params):
    del e
    p = params
    l1, l2 = p['layers']
    src = edge_index[0].reshape(NT * NCHUNK, CH)
    dst = edge_index[1].reshape(NT * NCHUNK, CH)

    w1 = jnp.concatenate([l1['A_w'], l1['B_w'], l1['D_w'], l1['E_w']], axis=1)
    b1 = jnp.concatenate([l1['A_b'], l1['B_b'], l1['D_b'], l1['E_b']])[None, :]
    w2 = jnp.concatenate([l2['A_w'], l2['B_w'], l2['D_w'], l2['E_w']], axis=1)
    b2 = jnp.concatenate([l2['A_b'], l2['B_b'], l2['D_b'], l2['E_b']])[None, :]

    prep = pl.pallas_call(
        _prep_body,
        out_shape=[jax.ShapeDtypeStruct((N, H), jnp.float32),
                   jax.ShapeDtypeStruct((N, H), jnp.float32),
                   jax.ShapeDtypeStruct((2, N, H), jnp.float32),
                   jax.ShapeDtypeStruct((N, H), jnp.float32),
                   jax.ShapeDtypeStruct((2, 64), jnp.float32)],
    )
    hemb, ah1, db1, et1, ce1 = prep(
        h, p['emb_h_w'], p['emb_h_b'][None, :], w1, b1,
        p['emb_e_w'], p['emb_e_b'][None, :], l1['C_w'], l1['C_b'][None, :])

    edge1 = _make_edge_kernel(True)
    enew1, nd1 = edge1(
        db1.reshape(2 * N, H), et1, ce1.reshape(2, 1, 64), src, dst)

    sbr = 4000
    stats1 = pl.pallas_call(
        _estats_body,
        grid=(E // sbr,),
        in_specs=[pl.BlockSpec((2, sbr, 64), lambda i: (0, i, 0))],
        out_specs=[pl.BlockSpec((2, 2, 64), lambda i: (0, 0, 0))],
        out_shape=[jax.ShapeDtypeStruct((2, 2, 64), jnp.float32)],
    )(enew1.reshape(2, E, 64))[0]

    mid = pl.pallas_call(
        _mid_body,
        out_shape=[jax.ShapeDtypeStruct((N, H), jnp.float32),
                   jax.ShapeDtypeStruct((N, H), jnp.float32),
                   jax.ShapeDtypeStruct((2, N, H), jnp.float32),
                   jax.ShapeDtypeStruct((N, H), jnp.float32),
                   jax.ShapeDtypeStruct((2, H), jnp.float32),
                   jax.ShapeDtypeStruct((1, H), jnp.float32)],
    )
    h1, ah2, db2, et2, ss, cb2 = mid(
        hemb, ah1, nd1.reshape(2, N, H), stats1,
        l1['bn_h_g'][None, :], l1['bn_h_b'][None, :],
        l1['bn_e_g'][None, :], l1['bn_e_b'][None, :],
        w2, b2, l2['C_w'], l2['C_b'][None, :],
        p['emb_e_w'], p['emb_e_b'][None, :])

    br = 4000
    ce2 = pl.pallas_call(
        _ce2_body,
        grid=(E // br,),
        in_specs=[pl.BlockSpec((2, br, 64), lambda i: (0, i, 0)),
                  pl.BlockSpec((2, H), lambda i: (0, 0)),
                  pl.BlockSpec((H, H), lambda i: (0, 0)),
                  pl.BlockSpec((1, H), lambda i: (0, 0))],
        out_specs=[pl.BlockSpec((2, br, 64), lambda i: (0, i, 0))],
        out_shape=[jax.ShapeDtypeStruct((2, E, 64), jnp.float32)],
    )(enew1.reshape(2, E, 64), ss, l2['C_w'], cb2)[0]

    edge2 = _make_edge_kernel(False)
    _, nd2 = edge2(
        db2.reshape(2 * N, H), et2, ce2.reshape(2 * E, 64), src, dst)

    nw = p['mlp_n']
    gw = p['mlp_g']
    final = pl.pallas_call(
        _final_body,
        out_shape=[jax.ShapeDtypeStruct((N, 3), jnp.float32),
                   jax.ShapeDtypeStruct((1, 3), jnp.float32)],
    )
    node_out, graph_out = final(
        h1, ah2, nd2.reshape(2, N, H),
        l2['bn_h_g'][None, :], l2['bn_h_b'][None, :],
        p['lstm_b_ih'][None, :], p['lstm_b_hh'][None, :],
        nw[0][0], nw[0][1][None, :], nw[1][0], nw[1][1][None, :],
        nw[2][0], nw[2][1][None, :],
        gw[0][0], gw[0][1][None, :], gw[1][0], gw[1][1][None, :],
        gw[2][0], gw[2][1][None, :])
    return node_out, graph_out


# speedup vs baseline: 1.6622x; 1.6622x over previous
import functools

import jax
import jax.numpy as jnp
from jax import lax
from jax.experimental import pallas as pl
from jax.experimental.pallas import tpu as pltpu
from jax.experimental.pallas import tpu_sc as plsc

N = 10000
E = 160000
H = 128
NT = 16
EPT = E // NT
CH = 80
HCH = CH // 2
NCHUNK = EPT // CH
ROWS_PT = 640


def _prep_body(hin, embw, embb, w1, b1, embew, embeb, c1w, c1b,
               hemb_o, ah_o, db_o, et_o, ce1_o):
    hemb = jnp.dot(hin[...], embw[...], preferred_element_type=jnp.float32) + embb[...]
    hemb_o[...] = hemb
    p = jnp.dot(hemb, w1[...], preferred_element_type=jnp.float32) + b1[...]
    ah = p[:, :H]
    bh = p[:, H:2 * H]
    dh = p[:, 2 * H:3 * H]
    eh = p[:, 3 * H:]
    ah_o[...] = ah
    db_o[0] = jnp.concatenate([dh[:, :64], bh[:, :64]], axis=1)
    db_o[1] = jnp.concatenate([dh[:, 64:], bh[:, 64:]], axis=1)
    et_o[...] = eh
    e0 = embew[...] + embeb[...]
    ce = jnp.dot(e0, c1w[...], preferred_element_type=jnp.float32) + c1b[...]
    ce1_o[...] = jnp.concatenate([ce[:, :64], ce[:, 64:]], axis=0)


def _mid_body(hemb, ah1, nd1, stats, bnh_g, bnh_b, bne_g, bne_b,
              w2, b2, c2w, c2b, embew, embeb,
              h1_o, ah2_o, db2_o, et2_o, ss_o, cb2_o):
    num = jnp.concatenate([nd1[0][:, :64], nd1[1][:, :64]], axis=1)
    den = jnp.concatenate([nd1[0][:, 64:], nd1[1][:, 64:]], axis=1)
    hn = ah1[...] + num / (den + 1e-6)
    m = jnp.mean(hn, axis=0, keepdims=True)
    v = jnp.mean((hn - m) * (hn - m), axis=0, keepdims=True)
    hn = (hn - m) * lax.rsqrt(v + 1e-5) * bnh_g[...] + bnh_b[...]
    h1 = hemb[...] + jnp.maximum(hn, 0.0)
    h1_o[...] = h1
    s1 = jnp.concatenate([stats[0][0:1, :], stats[1][0:1, :]], axis=1)
    q1 = jnp.concatenate([stats[0][1:2, :], stats[1][1:2, :]], axis=1)
    em = s1 / float(E)
    ev = q1 / float(E) - em * em
    scale = bne_g[...] * lax.rsqrt(ev + 1e-5)
    shift = bne_b[...] - em * scale
    ss_o[...] = jnp.concatenate([scale, shift], axis=0)
    e0 = embew[...] + embeb[...]
    cb2_o[...] = (jnp.dot(e0, c2w[...], preferred_element_type=jnp.float32)
                  + c2b[...])
    p2 = jnp.dot(h1, w2[...], preferred_element_type=jnp.float32) + b2[...]
    ah2_o[...] = p2[:, :H]
    bh = p2[:, H:2 * H]
    dh = p2[:, 2 * H:3 * H]
    eh = p2[:, 3 * H:]
    db2_o[0] = jnp.concatenate([dh[:, :64], bh[:, :64]], axis=1)
    db2_o[1] = jnp.concatenate([dh[:, 64:], bh[:, 64:]], axis=1)
    et2_o[...] = eh


def _estats_body(enew, st_o):
    i = pl.program_id(0)
    x = enew[...]
    s = jnp.sum(x, axis=1, keepdims=True)
    q = jnp.sum(x * x, axis=1, keepdims=True)
    cur = jnp.concatenate([s, q], axis=1)

    @pl.when(i == 0)
    def _():
        st_o[...] = cur

    @pl.when(i > 0)
    def _():
        st_o[...] = st_o[...] + cur


def _ce2_body(enew, ss, c2w, cb2, ce_o):
    x = jnp.concatenate([enew[0], enew[1]], axis=1)
    e1 = jnp.maximum(x * ss[0:1, :] + ss[1:2, :], 0.0)
    ce = jnp.dot(e1, c2w[...], preferred_element_type=jnp.float32) + cb2[...]
    ce_o[0] = ce[:, :64]
    ce_o[1] = ce[:, 64:]


def _final_body(h1, ah2, nd2, bnh_g, bnh_b, bih, bhh,
                nw0, nb0, nw1, nb1, nw2, nb2,
                gw0, gb0, gw1, gb1, gw2, gb2,
                node_o, graph_o):
    num = jnp.concatenate([nd2[0][:, :64], nd2[1][:, :64]], axis=1)
    den = jnp.concatenate([nd2[0][:, 64:], nd2[1][:, 64:]], axis=1)
    hn = ah2[...] + num / (den + 1e-6)
    m = jnp.mean(hn, axis=0, keepdims=True)
    v = jnp.mean((hn - m) * (hn - m), axis=0, keepdims=True)
    hn = (hn - m) * lax.rsqrt(v + 1e-5) * bnh_g[...] + bnh_b[...]
    h2 = h1[...] + jnp.maximum(hn, 0.0)
    gates = bih[...] + bhh[...]
    ig = gates[:, :H]
    gg = gates[:, 2 * H:3 * H]
    og = gates[:, 3 * H:]
    c = jax.nn.sigmoid(ig) * jnp.tanh(gg)
    q = jax.nn.sigmoid(og) * jnp.tanh(c)
    scores = jnp.sum(h2 * q, axis=1, keepdims=True)
    mx = jnp.max(scores, axis=0, keepdims=True)
    a = jnp.exp(scores - mx)
    alpha = a / jnp.sum(a, axis=0, keepdims=True)
    r = jnp.sum(alpha * h2, axis=0, keepdims=True)
    q_star = jnp.concatenate([q, r], axis=1)
    x = jnp.maximum(jnp.dot(h2, nw0[...], preferred_element_type=jnp.float32) + nb0[...], 0.0)
    x = jnp.maximum(jnp.dot(x, nw1[...], preferred_element_type=jnp.float32) + nb1[...], 0.0)
    node_o[...] = jnp.dot(x, nw2[...], preferred_element_type=jnp.float32) + nb2[...]
    g = jnp.maximum(jnp.dot(q_star, gw0[...], preferred_element_type=jnp.float32) + gb0[...], 0.0)
    g = jnp.maximum(jnp.dot(g, gw1[...], preferred_element_type=jnp.float32) + gb1[...], 0.0)
    graph_o[...] = jnp.dot(g, gw2[...], preferred_element_type=jnp.float32) + gb2[...]


_MESH = plsc.VectorSubcoreMesh(core_axis_name="c", subcore_axis_name="s",
                               num_cores=2, num_subcores=16)


def _sigmoid16(x):
    return 1.0 / (1.0 + jnp.exp(-x))


def _edge_kernel_body(first_layer, db, et, ce, srcr, dstr,
                      enew_hbm, nd_hbm,
                      sh_nd, src_b, dst_b, db_bufs, e_bufs, aux,
                      buf_ns, ce1_v, semd0, semd1, seme0, seme1,
                      sem3):
    ce_buf = aux
    enew_v = aux
    semd = (semd0, semd1)
    seme = (seme0, seme1)
    cid = lax.axis_index("c")
    sid = lax.axis_index("s")
    coff = cid * N
    ch0 = pl.multiple_of(cid * 64, 64)
    out_base = pl.multiple_of(jnp.minimum(sid * ROWS_PT, N - ROWS_PT), CH)

    @pl.loop(0, CH)
    def _zero(i):
        for g in range(H // 16):
            buf_ns[i, pl.ds(g * 16, 16)] = jnp.zeros((16,), jnp.float32)

    for k in range(ROWS_PT // CH):
        pltpu.sync_copy(buf_ns, sh_nd.at[pl.ds(out_base + k * CH, CH)])

    ce_vecs = None
    if first_layer:
        pltpu.sync_copy(ce.at[cid], ce1_v)
        ce_vecs = [ce1_v[0, pl.ds(g * 16, 16)] for g in range(4)]

    plsc.subcore_barrier()

    def _load_idx(c, slot):
        row = jnp.minimum(sid * NCHUNK + c, NT * NCHUNK - 1)
        pltpu.sync_copy(srcr.at[pl.ds(row, 1)], src_b.at[pl.ds(slot, 1)])
        pltpu.sync_copy(dstr.at[pl.ds(row, 1)], dst_b.at[pl.ds(slot, 1)])
        for g in range(CH // 16):
            sl = pl.ds(g * 16, 16)
            src_b[slot, sl] = src_b[slot, sl] + coff

    def _issue_gather(slot, h):
        idx_s = src_b.at[slot, pl.ds(h * HCH, HCH)]
        idx_d = dst_b.at[slot, pl.ds(h * HCH, HCH)]
        pltpu.async_copy(db.at[idx_s], db_bufs.at[h], semd[h])
        pltpu.async_copy(et.at[idx_d], e_bufs.at[h], seme[h])

    def _wait_gather(h):
        pltpu.make_async_copy(db.at[pl.ds(0, HCH)], db_bufs.at[h], semd[h]).wait()
        pltpu.make_async_copy(et.at[pl.ds(0, HCH)], e_bufs.at[h], seme[h]).wait()

    def _half_compute(h):
        hoff = h * HCH

        def _row(r):
            for g in range(4):
                sl = pl.ds(g * 16, 16)
                sh = pl.ds(64 + g * 16, 16)
                d = db_bufs[h, r, sl]
                b = db_bufs[h, r, sh]
                ev = e_bufs[h, r, pl.ds(ch0 + g * 16, 16)]
                if first_layer:
                    x = ce_vecs[g] + d + ev
                else:
                    x = ce_buf[hoff + r, sl] + d + ev
                s = _sigmoid16(x)
                buf_ns[hoff + r, sl] = s * b
                buf_ns[hoff + r, sh] = s
                if first_layer:
                    enew_v[hoff + r, sl] = x

        plsc.parallel_loop(0, HCH, unroll=2)(_row)

    def _do_chunk(c):
        cs = lax.rem(c, 2)
        ns = 1 - cs
        ebase = sid * EPT + c * CH
        _issue_gather(cs, 1)
        if not first_layer:
            pltpu.async_copy(ce.at[pl.ds(cid * E + ebase, CH)], ce_buf, sem3)
        _load_idx(c + 1, ns)
        if first_layer:
            @pl.when(c > 0)
            def _():
                pltpu.make_async_copy(
                    enew_v, enew_hbm.at[pl.ds(0, CH)], sem3).wait()
        else:
            pltpu.make_async_copy(
                ce.at[pl.ds(0, CH)], ce_buf, sem3).wait()
        _wait_gather(0)
        _half_compute(0)
        _issue_gather(ns, 0)
        _wait_gather(1)
        _half_compute(1)
        pltpu.sync_copy(buf_ns, sh_nd.at[dst_b.at[cs]], add=True)
        if first_layer:
            pltpu.async_copy(enew_v, enew_hbm.at[pl.ds(cid * E + ebase, CH)],
                             sem3)

    _load_idx(0, 0)
    _issue_gather(0, 0)

    @pl.loop(0, NCHUNK)
    def _chunks(c):
        _do_chunk(c)

    if first_layer:
        pltpu.make_async_copy(enew_v, enew_hbm.at[pl.ds(0, CH)], sem3).wait()

    _wait_gather(0)

    plsc.subcore_barrier()
    pltpu.sync_copy(sh_nd.at[pl.ds(out_base, ROWS_PT)],
                    nd_hbm.at[pl.ds(coff + out_base, ROWS_PT)])


def _make_edge_kernel(first_layer):
    outs = [jax.ShapeDtypeStruct((2 * E, 64), jnp.float32),
            jax.ShapeDtypeStruct((2 * N, H), jnp.float32)]
    scratch = [
        pltpu.VMEM_SHARED((N, H), jnp.float32),
        pltpu.VMEM((2, CH), jnp.int32),
        pltpu.VMEM((2, CH), jnp.int32),
        pltpu.VMEM((2, HCH, H), jnp.float32),
        pltpu.VMEM((2, HCH, H), jnp.float32),
        pltpu.VMEM((CH, 64), jnp.float32),
        pltpu.VMEM((CH, H), jnp.float32),
        pltpu.VMEM((1, 64), jnp.float32),
        pltpu.SemaphoreType.DMA,
        pltpu.SemaphoreType.DMA,
        pltpu.SemaphoreType.DMA,
        pltpu.SemaphoreType.DMA,
        pltpu.SemaphoreType.DMA,
    ]
    return pl.kernel(functools.partial(_edge_kernel_body, first_layer),
                     out_type=outs, mesh=_MESH, scratch_types=scratch)


def kernel(h, e, edge_index, params):
    del e
    p = params
    l1, l2 = p['layers']
    src = edge_index[0].reshape(NT * NCHUNK, CH)
    dst = edge_index[1].reshape(NT * NCHUNK, CH)

    w1 = jnp.concatenate([l1['A_w'], l1['B_w'], l1['D_w'], l1['E_w']], axis=1)
    b1 = jnp.concatenate([l1['A_b'], l1['B_b'], l1['D_b'], l1['E_b']])[None, :]
    w2 = jnp.concatenate([l2['A_w'], l2['B_w'], l2['D_w'], l2['E_w']], axis=1)
    b2 = jnp.concatenate([l2['A_b'], l2['B_b'], l2['D_b'], l2['E_b']])[None, :]

    prep = pl.pallas_call(
        _prep_body,
        out_shape=[jax.ShapeDtypeStruct((N, H), jnp.float32),
                   jax.ShapeDtypeStruct((N, H), jnp.float32),
                   jax.ShapeDtypeStruct((2, N, H), jnp.float32),
                   jax.ShapeDtypeStruct((N, H), jnp.float32),
                   jax.ShapeDtypeStruct((2, 64), jnp.float32)],
    )
    hemb, ah1, db1, et1, ce1 = prep(
        h, p['emb_h_w'], p['emb_h_b'][None, :], w1, b1,
        p['emb_e_w'], p['emb_e_b'][None, :], l1['C_w'], l1['C_b'][None, :])

    edge1 = _make_edge_kernel(True)
    enew1, nd1 = edge1(
        db1.reshape(2 * N, H), et1, ce1.reshape(2, 1, 64), src, dst)

    sbr = 4000
    stats1 = pl.pallas_call(
        _estats_body,
        grid=(E // sbr,),
        in_specs=[pl.BlockSpec((2, sbr, 64), lambda i: (0, i, 0))],
        out_specs=[pl.BlockSpec((2, 2, 64), lambda i: (0, 0, 0))],
        out_shape=[jax.ShapeDtypeStruct((2, 2, 64), jnp.float32)],
    )(enew1.reshape(2, E, 64))[0]

    mid = pl.pallas_call(
        _mid_body,
        out_shape=[jax.ShapeDtypeStruct((N, H), jnp.float32),
                   jax.ShapeDtypeStruct((N, H), jnp.float32),
                   jax.ShapeDtypeStruct((2, N, H), jnp.float32),
                   jax.ShapeDtypeStruct((N, H), jnp.float32),
                   jax.ShapeDtypeStruct((2, H), jnp.float32),
                   jax.ShapeDtypeStruct((1, H), jnp.float32)],
    )
    h1, ah2, db2, et2, ss, cb2 = mid(
        hemb, ah1, nd1.reshape(2, N, H), stats1,
        l1['bn_h_g'][None, :], l1['bn_h_b'][None, :],
        l1['bn_e_g'][None, :], l1['bn_e_b'][None, :],
        w2, b2, l2['C_w'], l2['C_b'][None, :],
        p['emb_e_w'], p['emb_e_b'][None, :])

    br = 4000
    ce2 = pl.pallas_call(
        _ce2_body,
        grid=(E // br,),
        in_specs=[pl.BlockSpec((2, br, 64), lambda i: (0, i, 0)),
                  pl.BlockSpec((2, H), lambda i: (0, 0)),
                  pl.BlockSpec((H, H), lambda i: (0, 0)),
                  pl.BlockSpec((1, H), lambda i: (0, 0))],
        out_specs=[pl.BlockSpec((2, br, 64), lambda i: (0, i, 0))],
        out_shape=[jax.ShapeDtypeStruct((2, E, 64), jnp.float32)],
    )(enew1.reshape(2, E, 64), ss, l2['C_w'], cb2)[0]

    edge2 = _make_edge_kernel(False)
    _, nd2 = edge2(
        db2.reshape(2 * N, H), et2, ce2.reshape(2 * E, 64), src, dst)

    nw = p['mlp_n']
    gw = p['mlp_g']
    final = pl.pallas_call(
        _final_body,
        out_shape=[jax.ShapeDtypeStruct((N, 3), jnp.float32),
                   jax.ShapeDtypeStruct((1, 3), jnp.float32)],
    )
    node_out, graph_out = final(
        h1, ah2, nd2.reshape(2, N, H),
        l2['bn_h_g'][None, :], l2['bn_h_b'][None, :],
        p['lstm_b_ih'][None, :], p['lstm_b_hh'][None, :],
        nw[0][0], nw[0][1][None, :], nw[1][0], nw[1][1][None, :],
        nw[2][0], nw[2][1][None, :],
        gw[0][0], gw[0][1][None, :], gw[1][0], gw[1][1][None, :],
        gw[2][0], gw[2][1][None, :])
    return node_out, graph_out


# speedup vs baseline: 1.6691x; 1.0042x over previous
import functools

import jax
import jax.numpy as jnp
from jax import lax
from jax.experimental import pallas as pl
from jax.experimental.pallas import tpu as pltpu
from jax.experimental.pallas import tpu_sc as plsc

N = 10000
E = 160000
H = 128
NT = 16
EPT = E // NT
CH = 80
HCH = CH // 2
NCHUNK = EPT // CH
ROWS_PT = 640


def _prep_body(hin, embw, embb, w1, b1, embew, embeb, c1w, c1b,
               hemb_o, ah_o, db_o, et_o, ce1_o):
    hemb = jnp.dot(hin[...], embw[...], preferred_element_type=jnp.float32) + embb[...]
    hemb_o[...] = hemb
    p = jnp.dot(hemb, w1[...], preferred_element_type=jnp.float32) + b1[...]
    ah = p[:, :H]
    bh = p[:, H:2 * H]
    dh = p[:, 2 * H:3 * H]
    eh = p[:, 3 * H:]
    ah_o[...] = ah
    db_o[0] = jnp.concatenate([dh[:, :64], bh[:, :64]], axis=1)
    db_o[1] = jnp.concatenate([dh[:, 64:], bh[:, 64:]], axis=1)
    et_o[...] = eh
    e0 = embew[...] + embeb[...]
    ce = jnp.dot(e0, c1w[...], preferred_element_type=jnp.float32) + c1b[...]
    ce1_o[...] = jnp.concatenate([ce[:, :64], ce[:, 64:]], axis=0)


def _mid_body(hemb, ah1, nd1, stats, bnh_g, bnh_b, bne_g, bne_b,
              w2, b2, c2w, c2b, embew, embeb,
              h1_o, ah2_o, db2_o, et2_o, ss_o, cb2_o):
    num = jnp.concatenate([nd1[0][:, :64], nd1[1][:, :64]], axis=1)
    den = jnp.concatenate([nd1[0][:, 64:], nd1[1][:, 64:]], axis=1)
    hn = ah1[...] + num / (den + 1e-6)
    m = jnp.mean(hn, axis=0, keepdims=True)
    v = jnp.mean((hn - m) * (hn - m), axis=0, keepdims=True)
    hn = (hn - m) * lax.rsqrt(v + 1e-5) * bnh_g[...] + bnh_b[...]
    h1 = hemb[...] + jnp.maximum(hn, 0.0)
    h1_o[...] = h1
    s1 = jnp.concatenate([stats[0][0:1, :], stats[1][0:1, :]], axis=1)
    q1 = jnp.concatenate([stats[0][1:2, :], stats[1][1:2, :]], axis=1)
    em = s1 / float(E)
    ev = q1 / float(E) - em * em
    scale = bne_g[...] * lax.rsqrt(ev + 1e-5)
    shift = bne_b[...] - em * scale
    ss_o[...] = jnp.concatenate([scale, shift], axis=0)
    e0 = embew[...] + embeb[...]
    cb2_o[...] = (jnp.dot(e0, c2w[...], preferred_element_type=jnp.float32)
                  + c2b[...])
    p2 = jnp.dot(h1, w2[...], preferred_element_type=jnp.float32) + b2[...]
    ah2_o[...] = p2[:, :H]
    bh = p2[:, H:2 * H]
    dh = p2[:, 2 * H:3 * H]
    eh = p2[:, 3 * H:]
    db2_o[0] = jnp.concatenate([dh[:, :64], bh[:, :64]], axis=1)
    db2_o[1] = jnp.concatenate([dh[:, 64:], bh[:, 64:]], axis=1)
    et2_o[...] = eh


def _estats_body(enew, st_o):
    i = pl.program_id(0)
    x = enew[...]
    s = jnp.sum(x, axis=1, keepdims=True)
    q = jnp.sum(x * x, axis=1, keepdims=True)
    cur = jnp.concatenate([s, q], axis=1)

    @pl.when(i == 0)
    def _():
        st_o[...] = cur

    @pl.when(i > 0)
    def _():
        st_o[...] = st_o[...] + cur


def _ce2_body(enew, ss, c2w, cb2, ce_o):
    x = jnp.concatenate([enew[0], enew[1]], axis=1)
    e1 = jnp.maximum(x * ss[0:1, :] + ss[1:2, :], 0.0)
    ce = jnp.dot(e1, c2w[...], preferred_element_type=jnp.float32) + cb2[...]
    ce_o[0] = ce[:, :64]
    ce_o[1] = ce[:, 64:]


def _final_body(h1, ah2, nd2, bnh_g, bnh_b, bih, bhh,
                nw0, nb0, nw1, nb1, nw2, nb2,
                gw0, gb0, gw1, gb1, gw2, gb2,
                node_o, graph_o):
    num = jnp.concatenate([nd2[0][:, :64], nd2[1][:, :64]], axis=1)
    den = jnp.concatenate([nd2[0][:, 64:], nd2[1][:, 64:]], axis=1)
    hn = ah2[...] + num / (den + 1e-6)
    m = jnp.mean(hn, axis=0, keepdims=True)
    v = jnp.mean((hn - m) * (hn - m), axis=0, keepdims=True)
    hn = (hn - m) * lax.rsqrt(v + 1e-5) * bnh_g[...] + bnh_b[...]
    h2 = h1[...] + jnp.maximum(hn, 0.0)
    gates = bih[...] + bhh[...]
    ig = gates[:, :H]
    gg = gates[:, 2 * H:3 * H]
    og = gates[:, 3 * H:]
    c = jax.nn.sigmoid(ig) * jnp.tanh(gg)
    q = jax.nn.sigmoid(og) * jnp.tanh(c)
    scores = jnp.sum(h2 * q, axis=1, keepdims=True)
    mx = jnp.max(scores, axis=0, keepdims=True)
    a = jnp.exp(scores - mx)
    alpha = a / jnp.sum(a, axis=0, keepdims=True)
    r = jnp.sum(alpha * h2, axis=0, keepdims=True)
    q_star = jnp.concatenate([q, r], axis=1)
    x = jnp.maximum(jnp.dot(h2, nw0[...], preferred_element_type=jnp.float32) + nb0[...], 0.0)
    x = jnp.maximum(jnp.dot(x, nw1[...], preferred_element_type=jnp.float32) + nb1[...], 0.0)
    node_o[...] = jnp.dot(x, nw2[...], preferred_element_type=jnp.float32) + nb2[...]
    g = jnp.maximum(jnp.dot(q_star, gw0[...], preferred_element_type=jnp.float32) + gb0[...], 0.0)
    g = jnp.maximum(jnp.dot(g, gw1[...], preferred_element_type=jnp.float32) + gb1[...], 0.0)
    graph_o[...] = jnp.dot(g, gw2[...], preferred_element_type=jnp.float32) + gb2[...]


_MESH = plsc.VectorSubcoreMesh(core_axis_name="c", subcore_axis_name="s",
                               num_cores=2, num_subcores=16)


def _sigmoid16(x):
    return 1.0 / (1.0 + jnp.exp(-x))


def _edge_kernel_body(first_layer, db, et, ce, srcr, dstr,
                      enew_hbm, nd_hbm,
                      sh_nd, src_b, dst_b, db_bufs, e_bufs, aux,
                      buf_ns, ce1_v, semd0, semd1, seme0, seme1,
                      sem3):
    ce_buf = aux
    enew_v = aux
    semd = (semd0, semd1)
    seme = (seme0, seme1)
    cid = lax.axis_index("c")
    sid = lax.axis_index("s")
    coff = cid * N
    ch0 = pl.multiple_of(cid * 64, 64)
    out_base = pl.multiple_of(jnp.minimum(sid * ROWS_PT, N - ROWS_PT), CH)

    @pl.loop(0, CH)
    def _zero(i):
        for g in range(H // 16):
            buf_ns[i, pl.ds(g * 16, 16)] = jnp.zeros((16,), jnp.float32)

    for k in range(ROWS_PT // CH):
        pltpu.sync_copy(buf_ns, sh_nd.at[pl.ds(out_base + k * CH, CH)])

    ce_vecs = None
    if first_layer:
        pltpu.sync_copy(ce.at[cid], ce1_v)
        ce_vecs = [ce1_v[0, pl.ds(g * 16, 16)] for g in range(4)]

    plsc.subcore_barrier()

    def _load_idx(c, slot):
        row = jnp.minimum(sid * NCHUNK + c, NT * NCHUNK - 1)
        pltpu.sync_copy(srcr.at[pl.ds(row, 1)], src_b.at[pl.ds(slot, 1)])
        pltpu.sync_copy(dstr.at[pl.ds(row, 1)], dst_b.at[pl.ds(slot, 1)])
        for g in range(CH // 16):
            sl = pl.ds(g * 16, 16)
            src_b[slot, sl] = src_b[slot, sl] + coff

    def _issue_gather(slot, h):
        idx_s = src_b.at[slot, pl.ds(h * HCH, HCH)]
        idx_d = dst_b.at[slot, pl.ds(h * HCH, HCH)]
        pltpu.async_copy(db.at[idx_s], db_bufs.at[h], semd[h])
        pltpu.async_copy(et.at[idx_d], e_bufs.at[h], seme[h])

    def _wait_gather(h):
        pltpu.make_async_copy(db.at[pl.ds(0, HCH)], db_bufs.at[h], semd[h]).wait()
        pltpu.make_async_copy(et.at[pl.ds(0, HCH)], e_bufs.at[h], seme[h]).wait()

    def _half_compute(h):
        hoff = h * HCH

        def _row(r):
            for g in range(4):
                sl = pl.ds(g * 16, 16)
                sh = pl.ds(64 + g * 16, 16)
                d = db_bufs[h, r, sl]
                b = db_bufs[h, r, sh]
                ev = e_bufs[h, r, pl.ds(ch0 + g * 16, 16)]
                if first_layer:
                    x = ce_vecs[g] + d + ev
                else:
                    x = ce_buf[hoff + r, sl] + d + ev
                s = _sigmoid16(x)
                buf_ns[hoff + r, sl] = s * b
                buf_ns[hoff + r, sh] = s
                if first_layer:
                    enew_v[hoff + r, sl] = x

        plsc.parallel_loop(0, HCH, unroll=4)(_row)

    def _do_chunk(c):
        cs = lax.rem(c, 2)
        ns = 1 - cs
        ebase = sid * EPT + c * CH
        _issue_gather(cs, 1)
        if not first_layer:
            pltpu.async_copy(ce.at[pl.ds(cid * E + ebase, CH)], ce_buf, sem3)
        _load_idx(c + 1, ns)
        if first_layer:
            @pl.when(c > 0)
            def _():
                pltpu.make_async_copy(
                    enew_v, enew_hbm.at[pl.ds(0, CH)], sem3).wait()
        else:
            pltpu.make_async_copy(
                ce.at[pl.ds(0, CH)], ce_buf, sem3).wait()
        _wait_gather(0)
        _half_compute(0)
        _issue_gather(ns, 0)
        _wait_gather(1)
        _half_compute(1)
        pltpu.sync_copy(buf_ns, sh_nd.at[dst_b.at[cs]], add=True)
        if first_layer:
            pltpu.async_copy(enew_v, enew_hbm.at[pl.ds(cid * E + ebase, CH)],
                             sem3)

    _load_idx(0, 0)
    _issue_gather(0, 0)

    @pl.loop(0, NCHUNK)
    def _chunks(c):
        _do_chunk(c)

    if first_layer:
        pltpu.make_async_copy(enew_v, enew_hbm.at[pl.ds(0, CH)], sem3).wait()

    _wait_gather(0)

    plsc.subcore_barrier()
    pltpu.sync_copy(sh_nd.at[pl.ds(out_base, ROWS_PT)],
                    nd_hbm.at[pl.ds(coff + out_base, ROWS_PT)])


def _make_edge_kernel(first_layer):
    outs = [jax.ShapeDtypeStruct((2 * E, 64), jnp.float32),
            jax.ShapeDtypeStruct((2 * N, H), jnp.float32)]
    scratch = [
        pltpu.VMEM_SHARED((N, H), jnp.float32),
        pltpu.VMEM((2, CH), jnp.int32),
        pltpu.VMEM((2, CH), jnp.int32),
        pltpu.VMEM((2, HCH, H), jnp.float32),
        pltpu.VMEM((2, HCH, H), jnp.float32),
        pltpu.VMEM((CH, 64), jnp.float32),
        pltpu.VMEM((CH, H), jnp.float32),
        pltpu.VMEM((1, 64), jnp.float32),
        pltpu.SemaphoreType.DMA,
        pltpu.SemaphoreType.DMA,
        pltpu.SemaphoreType.DMA,
        pltpu.SemaphoreType.DMA,
        pltpu.SemaphoreType.DMA,
    ]
    return pl.kernel(functools.partial(_edge_kernel_body, first_layer),
                     out_type=outs, mesh=_MESH, scratch_types=scratch)


def kernel(h, e, edge_index, params):
    del e
    p = params
    l1, l2 = p['layers']
    src = edge_index[0].reshape(NT * NCHUNK, CH)
    dst = edge_index[1].reshape(NT * NCHUNK, CH)

    w1 = jnp.concatenate([l1['A_w'], l1['B_w'], l1['D_w'], l1['E_w']], axis=1)
    b1 = jnp.concatenate([l1['A_b'], l1['B_b'], l1['D_b'], l1['E_b']])[None, :]
    w2 = jnp.concatenate([l2['A_w'], l2['B_w'], l2['D_w'], l2['E_w']], axis=1)
    b2 = jnp.concatenate([l2['A_b'], l2['B_b'], l2['D_b'], l2['E_b']])[None, :]

    prep = pl.pallas_call(
        _prep_body,
        out_shape=[jax.ShapeDtypeStruct((N, H), jnp.float32),
                   jax.ShapeDtypeStruct((N, H), jnp.float32),
                   jax.ShapeDtypeStruct((2, N, H), jnp.float32),
                   jax.ShapeDtypeStruct((N, H), jnp.float32),
                   jax.ShapeDtypeStruct((2, 64), jnp.float32)],
    )
    hemb, ah1, db1, et1, ce1 = prep(
        h, p['emb_h_w'], p['emb_h_b'][None, :], w1, b1,
        p['emb_e_w'], p['emb_e_b'][None, :], l1['C_w'], l1['C_b'][None, :])

    edge1 = _make_edge_kernel(True)
    enew1, nd1 = edge1(
        db1.reshape(2 * N, H), et1, ce1.reshape(2, 1, 64), src, dst)

    sbr = 4000
    stats1 = pl.pallas_call(
        _estats_body,
        grid=(E // sbr,),
        in_specs=[pl.BlockSpec((2, sbr, 64), lambda i: (0, i, 0))],
        out_specs=[pl.BlockSpec((2, 2, 64), lambda i: (0, 0, 0))],
        out_shape=[jax.ShapeDtypeStruct((2, 2, 64), jnp.float32)],
    )(enew1.reshape(2, E, 64))[0]

    mid = pl.pallas_call(
        _mid_body,
        out_shape=[jax.ShapeDtypeStruct((N, H), jnp.float32),
                   jax.ShapeDtypeStruct((N, H), jnp.float32),
                   jax.ShapeDtypeStruct((2, N, H), jnp.float32),
                   jax.ShapeDtypeStruct((N, H), jnp.float32),
                   jax.ShapeDtypeStruct((2, H), jnp.float32),
                   jax.ShapeDtypeStruct((1, H), jnp.float32)],
    )
    h1, ah2, db2, et2, ss, cb2 = mid(
        hemb, ah1, nd1.reshape(2, N, H), stats1,
        l1['bn_h_g'][None, :], l1['bn_h_b'][None, :],
        l1['bn_e_g'][None, :], l1['bn_e_b'][None, :],
        w2, b2, l2['C_w'], l2['C_b'][None, :],
        p['emb_e_w'], p['emb_e_b'][None, :])

    br = 4000
    ce2 = pl.pallas_call(
        _ce2_body,
        grid=(E // br,),
        in_specs=[pl.BlockSpec((2, br, 64), lambda i: (0, i, 0)),
                  pl.BlockSpec((2, H), lambda i: (0, 0)),
                  pl.BlockSpec((H, H), lambda i: (0, 0)),
                  pl.BlockSpec((1, H), lambda i: (0, 0))],
        out_specs=[pl.BlockSpec((2, br, 64), lambda i: (0, i, 0))],
        out_shape=[jax.ShapeDtypeStruct((2, E, 64), jnp.float32)],
    )(enew1.reshape(2, E, 64), ss, l2['C_w'], cb2)[0]

    edge2 = _make_edge_kernel(False)
    _, nd2 = edge2(
        db2.reshape(2 * N, H), et2, ce2.reshape(2 * E, 64), src, dst)

    nw = p['mlp_n']
    gw = p['mlp_g']
    final = pl.pallas_call(
        _final_body,
        out_shape=[jax.ShapeDtypeStruct((N, 3), jnp.float32),
                   jax.ShapeDtypeStruct((1, 3), jnp.float32)],
    )
    node_out, graph_out = final(
        h1, ah2, nd2.reshape(2, N, H),
        l2['bn_h_g'][None, :], l2['bn_h_b'][None, :],
        p['lstm_b_ih'][None, :], p['lstm_b_hh'][None, :],
        nw[0][0], nw[0][1][None, :], nw[1][0], nw[1][1][None, :],
        nw[2][0], nw[2][1][None, :],
        gw[0][0], gw[0][1][None, :], gw[1][0], gw[1][1][None, :],
        gw[2][0], gw[2][1][None, :])
    return node_out, graph_out
